# baseline (speedup 1.0000x reference)
import functools

import jax
import jax.numpy as jnp
from jax.experimental import pallas as pl
from jax.experimental.pallas import tpu as pltpu

_BM = 256
_BN = 512
_CC = 256
_LN = 128
_NEG = -1e30


def _mm_body(a_ref, b_ref, o_ref):
    k = pl.program_id(2)

    @pl.when(k == 0)
    def _():
        o_ref[...] = jnp.zeros_like(o_ref)

    o_ref[...] += jnp.dot(a_ref[...], b_ref[...],
                          preferred_element_type=jnp.float32)


def _matmul(a, b):
    m, k = a.shape
    k2, n = b.shape
    assert k == k2 and m % _BM == 0 and n % _BN == 0
    bk = min(512, k)
    assert k % bk == 0
    return pl.pallas_call(
        _mm_body,
        grid=(m // _BM, n // _BN, k // bk),
        in_specs=[
            pl.BlockSpec((_BM, bk), lambda i, j, kk: (i, kk)),
            pl.BlockSpec((bk, _BN), lambda i, j, kk: (kk, j)),
        ],
        out_specs=pl.BlockSpec((_BM, _BN), lambda i, j, kk: (i, j)),
        out_shape=jax.ShapeDtypeStruct((m, n), jnp.float32),
        compiler_params=pltpu.CompilerParams(
            dimension_semantics=("parallel", "parallel", "arbitrary")),
    )(a, b)


def _softmax_body(src_ref, dst_ref, elr_ref, alpha_ref, scr_ref, *,
                  ne, nh):
    scr_ref[...] = jnp.concatenate(
        [jnp.full((scr_ref.shape[0], nh), _NEG, jnp.float32),
         jnp.zeros((scr_ref.shape[0], nh), jnp.float32)], axis=1)

    def p1(e, _):
        s = src_ref[e]
        d = dst_ref[e]
        v = elr_ref[pl.ds(s, 1), 0:nh] + elr_ref[pl.ds(d, 1), nh:2 * nh]
        v = jnp.where(v > 0, v, 0.2 * v)
        alpha_ref[pl.ds(e, 1), :] = v
        scr_ref[pl.ds(d, 1), 0:nh] = jnp.maximum(
            scr_ref[pl.ds(d, 1), 0:nh], v)
        return 0

    jax.lax.fori_loop(0, ne, p1, 0)

    def p2(e, _):
        d = dst_ref[e]
        ex = jnp.exp(alpha_ref[pl.ds(e, 1), :]
                     - scr_ref[pl.ds(d, 1), 0:nh])
        alpha_ref[pl.ds(e, 1), :] = ex
        scr_ref[pl.ds(d, 1), nh:2 * nh] = (
            scr_ref[pl.ds(d, 1), nh:2 * nh] + ex)
        return 0

    jax.lax.fori_loop(0, ne, p2, 0)

    def p3(e, _):
        d = dst_ref[e]
        alpha_ref[pl.ds(e, 1), :] = (
            alpha_ref[pl.ds(e, 1), :] / scr_ref[pl.ds(d, 1), nh:2 * nh])
        return 0

    jax.lax.fori_loop(0, ne, p3, 0)


def _edge_softmax(elr, src, dst):
    np_, nh2 = elr.shape
    nh = nh2 // 2
    ne = src.shape[0]
    grid_spec = pltpu.PrefetchScalarGridSpec(
        num_scalar_prefetch=2,
        grid=(1,),
        in_specs=[pl.BlockSpec((np_, nh2), lambda i, *_: (0, 0))],
        out_specs=pl.BlockSpec((ne, nh), lambda i, *_: (0, 0)),
        scratch_shapes=[pltpu.VMEM((np_, nh2), jnp.float32)],
    )
    return pl.pallas_call(
        functools.partial(_softmax_body, ne=ne, nh=nh),
        grid_spec=grid_spec,
        out_shape=jax.ShapeDtypeStruct((ne, nh), jnp.float32),
    )(src, dst, elr)


def _agg_body(src_ref, dst_ref, ft_ref, ap_ref, o_ref, *, ne):
    o_ref[...] = jnp.zeros_like(o_ref)
    lanes = jax.lax.broadcasted_iota(jnp.int32, (1, _LN), 1)

    def p(e, _):
        s = src_ref[e]
        d = dst_ref[e]
        arow = ap_ref[pl.ds(e // _LN, 1), :]
        a = jnp.sum(arow * (lanes == e % _LN).astype(jnp.float32),
                    axis=1, keepdims=True)
        o_ref[pl.ds(d, 1), :] = (o_ref[pl.ds(d, 1), :]
                                 + a * ft_ref[pl.ds(s, 1), :])
        return 0

    jax.lax.fori_loop(0, ne, p, 0)


def _aggregate(ft, alpha_p, src, dst, heads, o):
    np_ = ft.shape[0]
    ne = src.shape[0]
    rows = alpha_p.shape[0] // 16
    cph = o // _CC
    grid_spec = pltpu.PrefetchScalarGridSpec(
        num_scalar_prefetch=2,
        grid=(heads * cph,),
        in_specs=[
            pl.BlockSpec((np_, _CC), lambda g, *_: (0, g)),
            pl.BlockSpec((rows, _LN), lambda g, *_: (g // cph, 0)),
        ],
        out_specs=pl.BlockSpec((np_, _CC), lambda g, *_: (0, g)),
    )
    return pl.pallas_call(
        functools.partial(_agg_body, ne=ne),
        grid_spec=grid_spec,
        out_shape=jax.ShapeDtypeStruct((np_, heads * o), jnp.float32),
    )(src, dst, ft, alpha_p)


def _addrelu_body(a_ref, b_ref, o_ref):
    o_ref[...] = jnp.maximum(a_ref[...] + b_ref[...], 0.0)


def _add_relu(a, b):
    m, n = a.shape
    bn = min(2048, n)
    return pl.pallas_call(
        _addrelu_body,
        grid=(m // _BM, n // bn),
        in_specs=[
            pl.BlockSpec((_BM, bn), lambda i, j: (i, j)),
            pl.BlockSpec((_BM, bn), lambda i, j: (i, j)),
        ],
        out_specs=pl.BlockSpec((_BM, bn), lambda i, j: (i, j)),
        out_shape=jax.ShapeDtypeStruct((m, n), jnp.float32),
    )(a, b)


def _fold(W, al, ar, heads, o):
    k = W.shape[0]
    wal = jnp.einsum('dho,ho->dh', W.reshape(k, heads, o), al[0])
    war = jnp.einsum('dho,ho->dh', W.reshape(k, heads, o), ar[0])
    blk = jnp.concatenate([wal, war], axis=1)
    return jnp.pad(blk, ((0, 0), (0, _BN - 2 * heads)))


def _pack_alpha(alpha, ne):
    rows = (-(-ne // _LN) + 7) // 8 * 8
    ap = jnp.pad(alpha, ((0, rows * _LN - ne), (0, 0)))
    return ap.T.reshape(16 * rows, _LN)


def _gat_layer(x, src, dst, W, al, ar, heads, o, res_W):
    cols = [W]
    if res_W is not None:
        cols.append(res_W)
    cols.append(_fold(W, al, ar, heads, o))
    waug = jnp.concatenate(cols, axis=1)
    out = _matmul(x, waug)
    ft = out[:, :heads * o]
    if res_W is not None:
        res = out[:, heads * o:2 * heads * o]
        base = 2 * heads * o
    else:
        res = x
        base = heads * o
    elr = out[:, base:base + 2 * heads]
    if heads < 16:
        elr = jnp.concatenate(
            [jnp.pad(elr[:, :heads], ((0, 0), (0, 16 - heads))),
             jnp.pad(elr[:, heads:], ((0, 0), (0, 16 - heads)))], axis=1)
    alpha = _edge_softmax(elr, src, dst)
    ap = _pack_alpha(alpha, src.shape[0])
    rst = _aggregate(ft, ap, src, dst, heads, o)
    return rst, res, alpha[:, :heads]


def kernel(feat, edge_index, W1, resW1, al1, ar1, W2, al2, ar2, W3, al3,
           ar3, W4, resW4, al4, ar4):
    n, d_in = feat.shape
    heads = al1.shape[1]
    o = al1.shape[2]
    src = edge_index[0]
    dst = edge_index[1]
    np_ = ((n + _BM - 1) // _BM) * _BM
    x = jnp.pad(feat, ((0, np_ - n), (0, 0)))

    rst, res, _ = _gat_layer(x, src, dst, W1, al1, ar1, heads, o, resW1)
    x = _add_relu(rst, res)
    rst, res, _ = _gat_layer(x, src, dst, W2, al2, ar2, heads, o, None)
    x = _add_relu(rst, res)
    rst, res, _ = _gat_layer(x, src, dst, W3, al3, ar3, heads, o, None)
    x = _add_relu(rst, res)
    rst, res, attn = _gat_layer(x, src, dst, W4, al4, ar4, 1, o, resW4)
    xo = _add_relu(rst, res)

    return (xo[:n], attn.reshape(-1, 1, 1), rst[:n])


# speedup vs baseline: 2.4624x; 2.4624x over previous
import functools

import jax
import jax.numpy as jnp
from jax.experimental import pallas as pl
from jax.experimental.pallas import tpu as pltpu

_BM = 256
_BN = 512
_CC = 128
_LN = 128
_NEG = -1e30


def _mm_body(a_ref, b_ref, o_ref):
    k = pl.program_id(2)

    @pl.when(k == 0)
    def _():
        o_ref[...] = jnp.zeros_like(o_ref)

    o_ref[...] += jnp.dot(a_ref[...], b_ref[...],
                          preferred_element_type=jnp.float32)


def _matmul(a, b):
    m, k = a.shape
    k2, n = b.shape
    assert k == k2 and m % _BM == 0 and n % _BN == 0
    bk = min(512, k)
    assert k % bk == 0
    return pl.pallas_call(
        _mm_body,
        grid=(m // _BM, n // _BN, k // bk),
        in_specs=[
            pl.BlockSpec((_BM, bk), lambda i, j, kk: (i, kk)),
            pl.BlockSpec((bk, _BN), lambda i, j, kk: (kk, j)),
        ],
        out_specs=pl.BlockSpec((_BM, _BN), lambda i, j, kk: (i, j)),
        out_shape=jax.ShapeDtypeStruct((m, n), jnp.float32),
        compiler_params=pltpu.CompilerParams(
            dimension_semantics=("parallel", "parallel", "arbitrary")),
    )(a, b)


def _softmax_body(src_ref, dst_ref, elr_ref, alpha_ref, sa_ref, sb_ref,
                  *, ne, nh):
    sa_ref[...] = jnp.concatenate(
        [jnp.full((sa_ref.shape[0], nh), _NEG, jnp.float32),
         jnp.zeros((sa_ref.shape[0], nh), jnp.float32)], axis=1)
    sb_ref[...] = sa_ref[...]

    def upd1(e, scr):
        s = src_ref[e]
        d = dst_ref[e]
        v = elr_ref[pl.ds(s, 1), 0:nh] + elr_ref[pl.ds(d, 1), nh:2 * nh]
        v = jnp.where(v > 0, v, 0.2 * v)
        alpha_ref[pl.ds(e, 1), :] = v
        scr[pl.ds(d, 1), 0:nh] = jnp.maximum(scr[pl.ds(d, 1), 0:nh], v)

    def p1(i, _):
        e = i * 8
        for j in range(8):
            upd1(e + j, sa_ref if j % 2 == 0 else sb_ref)
        return 0

    jax.lax.fori_loop(0, ne // 8, p1, 0)
    sa_ref[:, 0:nh] = jnp.maximum(sa_ref[:, 0:nh], sb_ref[:, 0:nh])

    def upd2(e, scr):
        d = dst_ref[e]
        ex = jnp.exp(alpha_ref[pl.ds(e, 1), :]
                     - sa_ref[pl.ds(d, 1), 0:nh])
        alpha_ref[pl.ds(e, 1), :] = ex
        scr[pl.ds(d, 1), nh:2 * nh] = scr[pl.ds(d, 1), nh:2 * nh] + ex

    def p2(i, _):
        e = i * 8
        for j in range(8):
            upd2(e + j, sa_ref if j % 2 == 0 else sb_ref)
        return 0

    jax.lax.fori_loop(0, ne // 8, p2, 0)
    sa_ref[:, nh:2 * nh] = (sa_ref[:, nh:2 * nh] + sb_ref[:, nh:2 * nh])

    def p3(i, _):
        e = i * 8
        for j in range(8):
            d = dst_ref[e + j]
            alpha_ref[pl.ds(e + j, 1), :] = (
                alpha_ref[pl.ds(e + j, 1), :]
                / sa_ref[pl.ds(d, 1), nh:2 * nh])
        return 0

    jax.lax.fori_loop(0, ne // 8, p3, 0)


def _edge_softmax(elr, src, dst):
    np_, nh2 = elr.shape
    nh = nh2 // 2
    ne = src.shape[0]
    assert ne % 8 == 0
    grid_spec = pltpu.PrefetchScalarGridSpec(
        num_scalar_prefetch=2,
        grid=(1,),
        in_specs=[pl.BlockSpec((np_, nh2), lambda i, *_: (0, 0))],
        out_specs=pl.BlockSpec((ne, nh), lambda i, *_: (0, 0)),
        scratch_shapes=[pltpu.VMEM((np_, nh2), jnp.float32),
                        pltpu.VMEM((np_, nh2), jnp.float32)],
    )
    return pl.pallas_call(
        functools.partial(_softmax_body, ne=ne, nh=nh),
        grid_spec=grid_spec,
        out_shape=jax.ShapeDtypeStruct((ne, nh), jnp.float32),
    )(src, dst, elr)


def _agg_body(src_ref, dst_ref, ft_ref, ap_ref, o_ref,
              p0_ref, p1_ref, p2_ref, p3_ref, *, ne):
    parts = (p0_ref, p1_ref, p2_ref, p3_ref)
    for pr in parts:
        pr[...] = jnp.zeros_like(pr)
    lanes = jax.lax.broadcasted_iota(jnp.int32, (1, _LN), 1)

    def upd(e, pr):
        s = src_ref[e]
        d = dst_ref[e]
        arow = ap_ref[pl.ds(e // _LN, 1), :]
        a = jnp.sum(arow * (lanes == e % _LN).astype(jnp.float32),
                    axis=1, keepdims=True)
        pr[pl.ds(d, 1), :] = pr[pl.ds(d, 1), :] + a * ft_ref[pl.ds(s, 1), :]

    def p(i, _):
        e = i * 8
        for j in range(8):
            upd(e + j, parts[j % 4])
        return 0

    jax.lax.fori_loop(0, ne // 8, p, 0)
    o_ref[...] = ((p0_ref[...] + p1_ref[...])
                  + (p2_ref[...] + p3_ref[...]))


def _aggregate(ft, alpha_p, src, dst, heads, o):
    np_ = ft.shape[0]
    ne = src.shape[0]
    assert ne % 8 == 0
    rows = alpha_p.shape[0] // 16
    cph = o // _CC
    grid_spec = pltpu.PrefetchScalarGridSpec(
        num_scalar_prefetch=2,
        grid=(heads * cph,),
        in_specs=[
            pl.BlockSpec((np_, _CC), lambda g, *_: (0, g)),
            pl.BlockSpec((rows, _LN), lambda g, *_: (g // cph, 0)),
        ],
        out_specs=pl.BlockSpec((np_, _CC), lambda g, *_: (0, g)),
        scratch_shapes=[pltpu.VMEM((np_, _CC), jnp.float32)
                        for _ in range(4)],
    )
    return pl.pallas_call(
        functools.partial(_agg_body, ne=ne),
        grid_spec=grid_spec,
        out_shape=jax.ShapeDtypeStruct((np_, heads * o), jnp.float32),
    )(src, dst, ft, alpha_p)


def _addrelu_body(a_ref, b_ref, o_ref):
    o_ref[...] = jnp.maximum(a_ref[...] + b_ref[...], 0.0)


def _add_relu(a, b):
    m, n = a.shape
    bn = min(2048, n)
    return pl.pallas_call(
        _addrelu_body,
        grid=(m // _BM, n // bn),
        in_specs=[
            pl.BlockSpec((_BM, bn), lambda i, j: (i, j)),
            pl.BlockSpec((_BM, bn), lambda i, j: (i, j)),
        ],
        out_specs=pl.BlockSpec((_BM, bn), lambda i, j: (i, j)),
        out_shape=jax.ShapeDtypeStruct((m, n), jnp.float32),
    )(a, b)


def _fold(W, al, ar, heads, o):
    k = W.shape[0]
    wal = jnp.einsum('dho,ho->dh', W.reshape(k, heads, o), al[0])
    war = jnp.einsum('dho,ho->dh', W.reshape(k, heads, o), ar[0])
    blk = jnp.concatenate([wal, war], axis=1)
    return jnp.pad(blk, ((0, 0), (0, _BN - 2 * heads)))


def _pack_alpha(alpha, ne):
    rows = (-(-ne // _LN) + 7) // 8 * 8
    ap = jnp.pad(alpha, ((0, rows * _LN - ne), (0, 0)))
    return ap.T.reshape(16 * rows, _LN)


def _gat_layer(x, src, dst, W, al, ar, heads, o, res_W):
    cols = [W]
    if res_W is not None:
        cols.append(res_W)
    cols.append(_fold(W, al, ar, heads, o))
    waug = jnp.concatenate(cols, axis=1)
    out = _matmul(x, waug)
    ft = out[:, :heads * o]
    if res_W is not None:
        res = out[:, heads * o:2 * heads * o]
        base = 2 * heads * o
    else:
        res = x
        base = heads * o
    elr = out[:, base:base + 2 * heads]
    if heads < 16:
        elr = jnp.concatenate(
            [jnp.pad(elr[:, :heads], ((0, 0), (0, 16 - heads))),
             jnp.pad(elr[:, heads:], ((0, 0), (0, 16 - heads)))], axis=1)
    alpha = _edge_softmax(elr, src, dst)
    ap = _pack_alpha(alpha, src.shape[0])
    rst = _aggregate(ft, ap, src, dst, heads, o)
    return rst, res, alpha[:, :heads]


def kernel(feat, edge_index, W1, resW1, al1, ar1, W2, al2, ar2, W3, al3,
           ar3, W4, resW4, al4, ar4):
    n, d_in = feat.shape
    heads = al1.shape[1]
    o = al1.shape[2]
    src = edge_index[0]
    dst = edge_index[1]
    np_ = ((n + _BM - 1) // _BM) * _BM
    x = jnp.pad(feat, ((0, np_ - n), (0, 0)))

    rst, res, _ = _gat_layer(x, src, dst, W1, al1, ar1, heads, o, resW1)
    x = _add_relu(rst, res)
    rst, res, _ = _gat_layer(x, src, dst, W2, al2, ar2, heads, o, None)
    x = _add_relu(rst, res)
    rst, res, _ = _gat_layer(x, src, dst, W3, al3, ar3, heads, o, None)
    x = _add_relu(rst, res)
    rst, res, attn = _gat_layer(x, src, dst, W4, al4, ar4, 1, o, resW4)
    xo = _add_relu(rst, res)

    return (xo[:n], attn.reshape(-1, 1, 1), rst[:n])


# speedup vs baseline: 3.0040x; 1.2199x over previous
import functools

import jax
import jax.numpy as jnp
from jax.experimental import pallas as pl
from jax.experimental.pallas import tpu as pltpu

_BM = 256
_BN = 512
_CC = 128
_LN = 128
_NEG = -1e30


def _mm_body(a_ref, b_ref, o_ref):
    k = pl.program_id(2)

    @pl.when(k == 0)
    def _():
        o_ref[...] = jnp.zeros_like(o_ref)

    o_ref[...] += jnp.dot(a_ref[...], b_ref[...],
                          preferred_element_type=jnp.float32)


def _matmul(a, b):
    m, k = a.shape
    k2, n = b.shape
    assert k == k2 and m % _BM == 0 and n % _BN == 0
    bk = min(512, k)
    assert k % bk == 0
    return pl.pallas_call(
        _mm_body,
        grid=(m // _BM, n // _BN, k // bk),
        in_specs=[
            pl.BlockSpec((_BM, bk), lambda i, j, kk: (i, kk)),
            pl.BlockSpec((bk, _BN), lambda i, j, kk: (kk, j)),
        ],
        out_specs=pl.BlockSpec((_BM, _BN), lambda i, j, kk: (i, j)),
        out_shape=jax.ShapeDtypeStruct((m, n), jnp.float32),
        compiler_params=pltpu.CompilerParams(
            dimension_semantics=("parallel", "parallel", "arbitrary")),
    )(a, b)


def _softmax_body(src_ref, dst_ref, elr_ref, alpha_ref, sa_ref, sb_ref,
                  *, ne, nh):
    sa_ref[...] = jnp.concatenate(
        [jnp.full((sa_ref.shape[0], nh), _NEG, jnp.float32),
         jnp.zeros((sa_ref.shape[0], nh), jnp.float32)], axis=1)
    sb_ref[...] = sa_ref[...]

    def upd1(e, scr):
        s = src_ref[e]
        d = dst_ref[e]
        v = elr_ref[pl.ds(s, 1), 0:nh] + elr_ref[pl.ds(d, 1), nh:2 * nh]
        v = jnp.where(v > 0, v, 0.2 * v)
        alpha_ref[pl.ds(e, 1), :] = v
        scr[pl.ds(d, 1), 0:nh] = jnp.maximum(scr[pl.ds(d, 1), 0:nh], v)

    def p1(i, _):
        e = i * 16
        for j in range(16):
            upd1(e + j, sa_ref if j % 2 == 0 else sb_ref)
        return 0

    jax.lax.fori_loop(0, ne // 16, p1, 0)
    sa_ref[:, 0:nh] = jnp.maximum(sa_ref[:, 0:nh], sb_ref[:, 0:nh])

    def upd2(e, scr):
        d = dst_ref[e]
        ex = jnp.exp(alpha_ref[pl.ds(e, 1), :]
                     - sa_ref[pl.ds(d, 1), 0:nh])
        alpha_ref[pl.ds(e, 1), :] = ex
        scr[pl.ds(d, 1), nh:2 * nh] = scr[pl.ds(d, 1), nh:2 * nh] + ex

    def p2(i, _):
        e = i * 16
        for j in range(16):
            upd2(e + j, sa_ref if j % 2 == 0 else sb_ref)
        return 0

    jax.lax.fori_loop(0, ne // 16, p2, 0)
    sa_ref[:, nh:2 * nh] = (sa_ref[:, nh:2 * nh] + sb_ref[:, nh:2 * nh])

    def p3(i, _):
        e = i * 16
        for j in range(16):
            d = dst_ref[e + j]
            alpha_ref[pl.ds(e + j, 1), :] = (
                alpha_ref[pl.ds(e + j, 1), :]
                / sa_ref[pl.ds(d, 1), nh:2 * nh])
        return 0

    jax.lax.fori_loop(0, ne // 16, p3, 0)


def _edge_softmax(elr, src, dst):
    np_, nh2 = elr.shape
    nh = nh2 // 2
    ne = src.shape[0]
    assert ne % 16 == 0
    grid_spec = pltpu.PrefetchScalarGridSpec(
        num_scalar_prefetch=2,
        grid=(1,),
        in_specs=[pl.BlockSpec((np_, nh2), lambda i, *_: (0, 0))],
        out_specs=pl.BlockSpec((ne, nh), lambda i, *_: (0, 0)),
        scratch_shapes=[pltpu.VMEM((np_, nh2), jnp.float32),
                        pltpu.VMEM((np_, nh2), jnp.float32)],
    )
    return pl.pallas_call(
        functools.partial(_softmax_body, ne=ne, nh=nh),
        grid_spec=grid_spec,
        out_shape=jax.ShapeDtypeStruct((ne, nh), jnp.float32),
    )(src, dst, elr)


def _agg_body(src_ref, dst_ref, ft_ref, ap_ref, o_ref,
              p0_ref, p1_ref, p2_ref, p3_ref, *, ne):
    parts = (p0_ref, p1_ref, p2_ref, p3_ref)
    for pr in parts:
        pr[...] = jnp.zeros_like(pr)
    lanes = jax.lax.broadcasted_iota(jnp.int32, (1, _LN), 1)

    def upd(e, pr):
        s = src_ref[e]
        d = dst_ref[e]
        arow = ap_ref[pl.ds(e // _LN, 1), :]
        a = jnp.sum(arow * (lanes == e % _LN).astype(jnp.float32),
                    axis=1, keepdims=True)
        pr[pl.ds(d, 1), :] = pr[pl.ds(d, 1), :] + a * ft_ref[pl.ds(s, 1), :]

    def p(i, _):
        e = i * 16
        for j in range(16):
            upd(e + j, parts[j % 4])
        return 0

    jax.lax.fori_loop(0, ne // 16, p, 0)
    o_ref[...] = ((p0_ref[...] + p1_ref[...])
                  + (p2_ref[...] + p3_ref[...]))


def _aggregate(ft, alpha_p, src, dst, heads, o):
    np_ = ft.shape[0]
    ne = src.shape[0]
    assert ne % 16 == 0
    rows = alpha_p.shape[0] // 16
    cph = o // _CC
    grid_spec = pltpu.PrefetchScalarGridSpec(
        num_scalar_prefetch=2,
        grid=(heads * cph,),
        in_specs=[
            pl.BlockSpec((np_, _CC), lambda g, *_: (0, g)),
            pl.BlockSpec((rows, _LN), lambda g, *_: (g // cph, 0)),
        ],
        out_specs=pl.BlockSpec((np_, _CC), lambda g, *_: (0, g)),
        scratch_shapes=[pltpu.VMEM((np_, _CC), jnp.float32)
                        for _ in range(4)],
    )
    return pl.pallas_call(
        functools.partial(_agg_body, ne=ne),
        grid_spec=grid_spec,
        out_shape=jax.ShapeDtypeStruct((np_, heads * o), jnp.float32),
    )(src, dst, ft, alpha_p)


def _addrelu_body(a_ref, b_ref, o_ref):
    o_ref[...] = jnp.maximum(a_ref[...] + b_ref[...], 0.0)


def _add_relu(a, b):
    m, n = a.shape
    bn = min(2048, n)
    return pl.pallas_call(
        _addrelu_body,
        grid=(m // _BM, n // bn),
        in_specs=[
            pl.BlockSpec((_BM, bn), lambda i, j: (i, j)),
            pl.BlockSpec((_BM, bn), lambda i, j: (i, j)),
        ],
        out_specs=pl.BlockSpec((_BM, bn), lambda i, j: (i, j)),
        out_shape=jax.ShapeDtypeStruct((m, n), jnp.float32),
    )(a, b)


def _fold(W, al, ar, heads, o):
    k = W.shape[0]
    wal = jnp.einsum('dho,ho->dh', W.reshape(k, heads, o), al[0])
    war = jnp.einsum('dho,ho->dh', W.reshape(k, heads, o), ar[0])
    blk = jnp.concatenate([wal, war], axis=1)
    return jnp.pad(blk, ((0, 0), (0, _BN - 2 * heads)))


def _pack_alpha(alpha, ne):
    rows = (-(-ne // _LN) + 7) // 8 * 8
    ap = jnp.pad(alpha, ((0, rows * _LN - ne), (0, 0)))
    return ap.T.reshape(16 * rows, _LN)


def _gat_layer(x, src, dst, W, al, ar, heads, o, res_W):
    cols = [W]
    if res_W is not None:
        cols.append(res_W)
    cols.append(_fold(W, al, ar, heads, o))
    waug = jnp.concatenate(cols, axis=1)
    out = _matmul(x, waug)
    ft = out[:, :heads * o]
    if res_W is not None:
        res = out[:, heads * o:2 * heads * o]
        base = 2 * heads * o
    else:
        res = x
        base = heads * o
    elr = out[:, base:base + 2 * heads]
    if heads < 16:
        elr = jnp.concatenate(
            [jnp.pad(elr[:, :heads], ((0, 0), (0, 16 - heads))),
             jnp.pad(elr[:, heads:], ((0, 0), (0, 16 - heads)))], axis=1)
    alpha = _edge_softmax(elr, src, dst)
    ap = _pack_alpha(alpha, src.shape[0])
    rst = _aggregate(ft, ap, src, dst, heads, o)
    return rst, res, alpha[:, :heads]


def kernel(feat, edge_index, W1, resW1, al1, ar1, W2, al2, ar2, W3, al3,
           ar3, W4, resW4, al4, ar4):
    n, d_in = feat.shape
    heads = al1.shape[1]
    o = al1.shape[2]
    src = edge_index[0]
    dst = edge_index[1]
    np_ = ((n + _BM - 1) // _BM) * _BM
    x = jnp.pad(feat, ((0, np_ - n), (0, 0)))

    rst, res, _ = _gat_layer(x, src, dst, W1, al1, ar1, heads, o, resW1)
    x = _add_relu(rst, res)
    rst, res, _ = _gat_layer(x, src, dst, W2, al2, ar2, heads, o, None)
    x = _add_relu(rst, res)
    rst, res, _ = _gat_layer(x, src, dst, W3, al3, ar3, heads, o, None)
    x = _add_relu(rst, res)
    rst, res, attn = _gat_layer(x, src, dst, W4, al4, ar4, 1, o, resW4)
    xo = _add_relu(rst, res)

    return (xo[:n], attn.reshape(-1, 1, 1), rst[:n])
